# Initial kernel scaffold; baseline (speedup 1.0000x reference)
import jax
import jax.numpy as jnp
from jax.experimental import pallas as pl


def kernel(x, A_eff, w_conv, b_conv, gamma, beta):
    raise NotImplementedError("write your pallas kernel here")


# speedup vs baseline: 1.3093x; 1.3093x over previous
import jax
import jax.numpy as jnp
from jax.experimental import pallas as pl
from jax.experimental.pallas import tpu as pltpu

_S = 3
_V = 25
_GF = 5
_GL = _GF * _V
_NB = 4


def _pack_tv(a):
    *lead, T, V = a.shape
    G = -(-T // _GF)
    lead_pad = [(0, 0)] * len(lead)
    a = jnp.pad(a, lead_pad + [(0, G * _GF - T), (0, 0)])
    a = a.reshape(*lead, G, _GL)
    a = jnp.pad(a, lead_pad + [(0, 0), (0, 128 - _GL)])
    return a.reshape(*lead, G * 128)


def _msg_kernel(xp_ref, b_ref, w_ref, bias_ref, mask_ref,
                y_ref, s1_ref, s2_ref, agg_ref):
    NB, C, L = xp_ref.shape
    X = xp_ref[...].reshape(NB * C, L)
    for s in range(_S):
        for c in range(L // 256):
            sl = slice(256 * c, 256 * (c + 1))
            agg_ref[s, :, sl] = jnp.dot(
                X[:, sl], b_ref[s],
                preferred_element_type=jnp.float32).astype(jnp.bfloat16)
    w = w_ref[...]
    bias = bias_ref[...]
    mask = mask_ref[...]
    for n in range(NB):
        a = jnp.concatenate(
            [agg_ref[s, C * n:C * (n + 1), :] for s in range(_S)], axis=0)
        y = jnp.dot(w, a, preferred_element_type=jnp.float32) + bias
        y_ref[n] = y
        ym = y * mask
        s1_ref[n] = jnp.sum(ym, axis=1, keepdims=True)
        s2_ref[n] = jnp.sum(ym * y, axis=1, keepdims=True)


def _bn_kernel(y_ref, sc_ref, sh_ref, o_ref):
    NB, O, TV = o_ref.shape
    sc = sc_ref[...]
    sh = sh_ref[...]
    n_out_groups = -(-TV // _GL)
    for n in range(NB):
        z = jnp.maximum(y_ref[n] * sc + sh, 0.0)
        for g in range(n_out_groups):
            w = min(_GL, TV - _GL * g)
            o_ref[n, :, _GL * g:_GL * g + w] = z[:, 128 * g:128 * g + w]


def kernel(x, A_eff, w_conv, b_conv, gamma, beta):
    N, C, V, T = x.shape
    S = _S
    O = w_conv.shape[0]
    G = -(-T // _GF)
    if G % 2:
        G += 1
    L = G * 128

    xt = jnp.transpose(x, (0, 1, 3, 2))
    xt = jnp.pad(xt, ((0, 0), (0, 0), (0, G * _GF - T), (0, 0)))
    xp = _pack_tv(xt[..., :G * _GF, :]).astype(jnp.bfloat16)

    A3 = A_eff.reshape(S, V, V)
    AT = jnp.transpose(A3, (0, 2, 1)).astype(jnp.float32)
    B = jnp.zeros((S, 256, 256), jnp.float32)
    for j in range(2 * _GF):
        r = 128 * (j // _GF) + _V * (j % _GF)
        B = B.at[:, r:r + _V, r:r + _V].set(AT)
    B = B.astype(jnp.bfloat16)

    Wm = w_conv.astype(jnp.bfloat16)
    b2 = b_conv.reshape(O, 1).astype(jnp.float32)
    mask = _pack_tv(jnp.ones((1, T, V), jnp.float32))
    mask = jnp.pad(mask, ((0, 0), (0, L - mask.shape[-1])))

    y_pre, s1, s2 = pl.pallas_call(
        _msg_kernel,
        out_shape=(jax.ShapeDtypeStruct((N, O, L), jnp.float32),
                   jax.ShapeDtypeStruct((N, O, 1), jnp.float32),
                   jax.ShapeDtypeStruct((N, O, 1), jnp.float32)),
        grid=(N // _NB,),
        in_specs=[pl.BlockSpec((_NB, C, L), lambda i: (i, 0, 0)),
                  pl.BlockSpec((S, 256, 256), lambda i: (0, 0, 0)),
                  pl.BlockSpec((O, S * C), lambda i: (0, 0)),
                  pl.BlockSpec((O, 1), lambda i: (0, 0)),
                  pl.BlockSpec((1, L), lambda i: (0, 0))],
        out_specs=(pl.BlockSpec((_NB, O, L), lambda i: (i, 0, 0)),
                   pl.BlockSpec((_NB, O, 1), lambda i: (i, 0, 0)),
                   pl.BlockSpec((_NB, O, 1), lambda i: (i, 0, 0))),
        scratch_shapes=[pltpu.VMEM((S, _NB * C, L), jnp.bfloat16)],
        compiler_params=pltpu.CompilerParams(
            dimension_semantics=("parallel",),
            vmem_limit_bytes=64 * 1024 * 1024),
    )(xp, B, Wm, b2, mask)

    cnt = float(N * T * V)
    mu = jnp.sum(s1[:, :, 0], axis=0) / cnt
    ex2 = jnp.sum(s2[:, :, 0], axis=0) / cnt
    var = jnp.maximum(ex2 - mu * mu, 0.0)
    inv = jax.lax.rsqrt(var + 1e-5)
    scale = (gamma * inv).reshape(O, 1).astype(jnp.float32)
    shift = (beta - mu * gamma * inv).reshape(O, 1).astype(jnp.float32)

    out = pl.pallas_call(
        _bn_kernel,
        out_shape=jax.ShapeDtypeStruct((N, O, T * V), jnp.float32),
        grid=(N // _NB,),
        in_specs=[pl.BlockSpec((_NB, O, L), lambda i: (i, 0, 0)),
                  pl.BlockSpec((O, 1), lambda i: (0, 0)),
                  pl.BlockSpec((O, 1), lambda i: (0, 0))],
        out_specs=pl.BlockSpec((_NB, O, T * V), lambda i: (i, 0, 0)),
        compiler_params=pltpu.CompilerParams(
            dimension_semantics=("parallel",)),
    )(y_pre, scale, shift)

    return out.reshape(N, O, T, V)


# speedup vs baseline: 1.4835x; 1.1330x over previous
import jax
import jax.numpy as jnp
from jax.experimental import pallas as pl
from jax.experimental.pallas import tpu as pltpu

_S = 3
_V = 25
_GF = 5
_GL = _GF * _V
_NB = 4


def _pack_lanes_const(a):
    _, TV = a.shape
    G = -(-TV // _GL)
    a = jnp.pad(a, ((0, 0), (0, G * _GL - TV)))
    a = a.reshape(1, G, _GL)
    a = jnp.pad(a, ((0, 0), (0, 0), (0, 128 - _GL)))
    return a.reshape(1, G * 128)


def _msg_kernel(xt_ref, b_ref, w_ref, bias_ref, mask_ref,
                y_ref, s1_ref, s2_ref, xp_ref, agg_ref):
    NB, C, TV = xt_ref.shape
    L = xp_ref.shape[-1]
    X = xt_ref[...].reshape(NB * C, TV)
    ngo = -(-TV // _GL)
    for g in range(ngo):
        w = min(_GL, TV - _GL * g)
        xp_ref[:, 128 * g:128 * g + w] = \
            X[:, _GL * g:_GL * g + w].astype(jnp.bfloat16)
        xp_ref[:, 128 * g + w:128 * (g + 1)] = \
            jnp.zeros((NB * C, 128 - w), jnp.bfloat16)
    for g in range(ngo, L // 128):
        xp_ref[:, 128 * g:128 * (g + 1)] = jnp.zeros((NB * C, 128),
                                                     jnp.bfloat16)
    Xp = xp_ref[...]
    for s in range(_S):
        for c in range(L // 256):
            sl = slice(256 * c, 256 * (c + 1))
            agg_ref[s, :, sl] = jnp.dot(
                Xp[:, sl], b_ref[s],
                preferred_element_type=jnp.float32).astype(jnp.bfloat16)
    w = w_ref[...]
    bias = bias_ref[...]
    mask = mask_ref[...]
    for n in range(NB):
        a = jnp.concatenate(
            [agg_ref[s, C * n:C * (n + 1), :] for s in range(_S)], axis=0)
        y = jnp.dot(w, a, preferred_element_type=jnp.float32) + bias
        y_ref[n] = y
        ym = y * mask
        s1_ref[n] = jnp.sum(ym, axis=1, keepdims=True)
        s2_ref[n] = jnp.sum(ym * y, axis=1, keepdims=True)


def _bn_kernel(y_ref, s1_ref, s2_ref, g_ref, be_ref, o_ref):
    NB, O, TV = o_ref.shape
    N = s1_ref.shape[0]
    cnt = float(N * TV)
    mu = jnp.sum(s1_ref[...], axis=0) / cnt
    ex2 = jnp.sum(s2_ref[...], axis=0) / cnt
    var = jnp.maximum(ex2 - mu * mu, 0.0)
    inv = jax.lax.rsqrt(var + 1e-5)
    gcol = jnp.transpose(g_ref[...])
    bcol = jnp.transpose(be_ref[...])
    sc = gcol * inv
    sh = bcol - mu * sc
    n_out_groups = -(-TV // _GL)
    for n in range(NB):
        z = jnp.maximum(y_ref[n] * sc + sh, 0.0)
        for g in range(n_out_groups):
            w = min(_GL, TV - _GL * g)
            o_ref[n, :, _GL * g:_GL * g + w] = z[:, 128 * g:128 * g + w]


def kernel(x, A_eff, w_conv, b_conv, gamma, beta):
    N, C, V, T = x.shape
    S = _S
    O = w_conv.shape[0]
    G = -(-T // _GF)
    if G % 2:
        G += 1
    L = G * 128
    TV = T * V

    xt = jnp.transpose(x, (0, 1, 3, 2)).reshape(N, C, TV)

    A3 = A_eff.reshape(S, V, V)
    AT = jnp.swapaxes(A3, 1, 2)
    B = jnp.einsum('ab,suv->saubv', jnp.eye(2 * _GF, dtype=A_eff.dtype), AT)
    B = B.reshape(S, 2 * _GL, 2 * _GL)
    B = jnp.pad(B.reshape(S, 2, _GL, 2 * _GL),
                ((0, 0), (0, 0), (0, 128 - _GL), (0, 0)))
    B = B.reshape(S, 256, 2, _GL)
    B = jnp.pad(B, ((0, 0), (0, 0), (0, 0), (0, 128 - _GL)))
    B = B.reshape(S, 256, 256).astype(jnp.bfloat16)

    Wm = w_conv.astype(jnp.bfloat16)
    b2 = b_conv.reshape(O, 1).astype(jnp.float32)
    mask = _pack_lanes_const(jnp.ones((1, TV), jnp.float32))
    mask = jnp.pad(mask, ((0, 0), (0, L - mask.shape[-1])))

    y_pre, s1, s2 = pl.pallas_call(
        _msg_kernel,
        out_shape=(jax.ShapeDtypeStruct((N, O, L), jnp.float32),
                   jax.ShapeDtypeStruct((N, O, 1), jnp.float32),
                   jax.ShapeDtypeStruct((N, O, 1), jnp.float32)),
        grid=(N // _NB,),
        in_specs=[pl.BlockSpec((_NB, C, TV), lambda i: (i, 0, 0)),
                  pl.BlockSpec((S, 256, 256), lambda i: (0, 0, 0)),
                  pl.BlockSpec((O, S * C), lambda i: (0, 0)),
                  pl.BlockSpec((O, 1), lambda i: (0, 0)),
                  pl.BlockSpec((1, L), lambda i: (0, 0))],
        out_specs=(pl.BlockSpec((_NB, O, L), lambda i: (i, 0, 0)),
                   pl.BlockSpec((_NB, O, 1), lambda i: (i, 0, 0)),
                   pl.BlockSpec((_NB, O, 1), lambda i: (i, 0, 0))),
        scratch_shapes=[pltpu.VMEM((_NB * C, L), jnp.bfloat16),
                        pltpu.VMEM((S, _NB * C, L), jnp.bfloat16)],
        compiler_params=pltpu.CompilerParams(
            dimension_semantics=("parallel",),
            vmem_limit_bytes=64 * 1024 * 1024),
    )(xt, B, Wm, b2, mask)

    out = pl.pallas_call(
        _bn_kernel,
        out_shape=jax.ShapeDtypeStruct((N, O, TV), jnp.float32),
        grid=(N // _NB,),
        in_specs=[pl.BlockSpec((_NB, O, L), lambda i: (i, 0, 0)),
                  pl.BlockSpec((N, O, 1), lambda i: (0, 0, 0)),
                  pl.BlockSpec((N, O, 1), lambda i: (0, 0, 0)),
                  pl.BlockSpec((1, O), lambda i: (0, 0)),
                  pl.BlockSpec((1, O), lambda i: (0, 0))],
        out_specs=pl.BlockSpec((_NB, O, TV), lambda i: (i, 0, 0)),
        compiler_params=pltpu.CompilerParams(
            dimension_semantics=("parallel",)),
    )(y_pre, s1, s2, gamma.reshape(1, O), beta.reshape(1, O))

    return out.reshape(N, O, T, V)
